# Initial kernel scaffold; baseline (speedup 1.0000x reference)
import jax
import jax.numpy as jnp
from jax.experimental import pallas as pl


def kernel(x, start, end, W1, b1, W2, b2):
    raise NotImplementedError("write your pallas kernel here")


# speedup vs baseline: 1.2494x; 1.2494x over previous
import jax
import jax.numpy as jnp
from jax.experimental import pallas as pl
from jax.experimental.pallas import tpu as pltpu
from jax.experimental.pallas import tpu_sc as plsc

N_NODES = 10000
N_EDGES = 320000
D_FEAT = 128
HIDDEN = 256

GATHER_WINDOW = 256
N_GATHER_BLOCKS = N_EDGES // GATHER_WINDOW

EDGE_BLOCK = 2560
N_EDGE_BLOCKS = N_EDGES // EDGE_BLOCK


def _sc_gather(x, idx2d):
    mesh = plsc.VectorSubcoreMesh(core_axis_name="core", subcore_axis_name="subcore")

    def run(x, idx2d):
        @pl.kernel(
            out_type=jax.ShapeDtypeStruct((N_EDGES, 2 * D_FEAT), jnp.float32),
            mesh=mesh,
        )
        def gather_kernel(x_hbm, i_hbm, o_hbm):
            def body(i_vmem, o_vmem):
                pltpu.sync_copy(x_hbm.at[i_vmem.at[0]], o_vmem)

            pltpu.emit_pipeline(
                body,
                grid=(2 * N_GATHER_BLOCKS,),
                in_specs=[
                    pl.BlockSpec((1, GATHER_WINDOW), index_map=lambda i: (0, i))
                ],
                out_specs=[
                    pl.BlockSpec(
                        (GATHER_WINDOW, D_FEAT),
                        index_map=lambda i: (i % N_GATHER_BLOCKS, i // N_GATHER_BLOCKS),
                    )
                ],
                core_axis_name=("core", "subcore"),
                dimension_semantics=(pltpu.PARALLEL,),
            )(i_hbm, o_hbm)

        return gather_kernel(x, idx2d)

    return run(x, idx2d)


def _tc_mlp(g, W1, b1, W2, b2):
    def body(g_ref, w1_ref, b1_ref, w2_ref, b2_ref, o_ref):
        h = jnp.dot(
            g_ref[...],
            w1_ref[...],
            preferred_element_type=jnp.float32,
            precision=jax.lax.Precision.HIGHEST,
        )
        h = jnp.maximum(h + b1_ref[...], 0.0)
        o_ref[...] = (
            jnp.dot(
                h,
                w2_ref[...],
                preferred_element_type=jnp.float32,
                precision=jax.lax.Precision.HIGHEST,
            )
            + b2_ref[...]
        )

    return pl.pallas_call(
        body,
        grid=(N_EDGE_BLOCKS,),
        in_specs=[
            pl.BlockSpec((EDGE_BLOCK, 2 * D_FEAT), lambda i: (i, 0)),
            pl.BlockSpec((2 * D_FEAT, HIDDEN), lambda i: (0, 0)),
            pl.BlockSpec((1, HIDDEN), lambda i: (0, 0)),
            pl.BlockSpec((HIDDEN, 1), lambda i: (0, 0)),
            pl.BlockSpec((1, 1), lambda i: (0, 0)),
        ],
        out_specs=pl.BlockSpec((EDGE_BLOCK, 1), lambda i: (i, 0)),
        out_shape=jax.ShapeDtypeStruct((N_EDGES, 1), jnp.float32),
    )(g, W1, b1, W2, b2)


def kernel(x, start, end, W1, b1, W2, b2):
    idx2d = jnp.concatenate([start, end]).reshape(1, 2 * N_EDGES)
    g = _sc_gather(x, idx2d)
    return _tc_mlp(g, W1, b1.reshape(1, HIDDEN), W2, b2.reshape(1, 1))


# speedup vs baseline: 2.9490x; 2.3603x over previous
import jax
import jax.numpy as jnp
from jax.experimental import pallas as pl
from jax.experimental.pallas import tpu as pltpu
from jax.experimental.pallas import tpu_sc as plsc

N_NODES = 10000
N_EDGES = 320000
D_FEAT = 128
HIDDEN = 256

GATHER_WINDOW = 256
N_GATHER_BLOCKS = N_EDGES // GATHER_WINDOW

EDGE_BLOCK = 2560
N_EDGE_BLOCKS = N_EDGES // EDGE_BLOCK


def _sc_gather(x, idx2d):
    mesh = plsc.VectorSubcoreMesh(core_axis_name="core", subcore_axis_name="subcore")

    @pl.kernel(
        out_type=jax.ShapeDtypeStruct((N_EDGES, 2 * D_FEAT), jnp.float32),
        mesh=mesh,
    )
    def gather_kernel(x_hbm, i_hbm, o_hbm):
        def body(i_vmem, o_vmem):
            pltpu.sync_copy(x_hbm.at[i_vmem.at[0]], o_vmem)

        pltpu.emit_pipeline(
            body,
            grid=(2 * N_GATHER_BLOCKS,),
            in_specs=[
                pl.BlockSpec((1, GATHER_WINDOW), index_map=lambda i: (0, i))
            ],
            out_specs=[
                pl.BlockSpec(
                    (GATHER_WINDOW, D_FEAT),
                    index_map=lambda i: (i % N_GATHER_BLOCKS, i // N_GATHER_BLOCKS),
                )
            ],
            core_axis_name=("core", "subcore"),
            dimension_semantics=(pltpu.PARALLEL,),
        )(i_hbm, o_hbm)

    return gather_kernel(x, idx2d)


def _tc_mlp(g, W1, b1, W2, b2):
    def body(g_ref, w1_ref, b1_ref, w2_ref, b2_ref, o_ref):
        h = jnp.dot(
            g_ref[...].astype(jnp.bfloat16),
            w1_ref[...],
            preferred_element_type=jnp.float32,
        )
        h = jnp.maximum(h + b1_ref[...], 0.0)
        o_ref[...] = (
            jnp.dot(
                h.astype(jnp.bfloat16),
                w2_ref[...],
                preferred_element_type=jnp.float32,
            )
            + b2_ref[...]
        )

    return pl.pallas_call(
        body,
        grid=(N_EDGE_BLOCKS,),
        in_specs=[
            pl.BlockSpec((EDGE_BLOCK, 2 * D_FEAT), lambda i: (i, 0)),
            pl.BlockSpec((2 * D_FEAT, HIDDEN), lambda i: (0, 0)),
            pl.BlockSpec((1, HIDDEN), lambda i: (0, 0)),
            pl.BlockSpec((HIDDEN, 1), lambda i: (0, 0)),
            pl.BlockSpec((1, 1), lambda i: (0, 0)),
        ],
        out_specs=pl.BlockSpec((EDGE_BLOCK, 1), lambda i: (i, 0)),
        out_shape=jax.ShapeDtypeStruct((N_EDGES, 1), jnp.float32),
    )(g, W1, b1, W2, b2)


def kernel(x, start, end, W1, b1, W2, b2):
    idx2d = jnp.concatenate([start, end]).reshape(1, 2 * N_EDGES)
    g = _sc_gather(x, idx2d)
    return _tc_mlp(
        g,
        W1.astype(jnp.bfloat16),
        b1.reshape(1, HIDDEN),
        W2.astype(jnp.bfloat16),
        b2.reshape(1, 1),
    )
